# baseline (speedup 1.0000x reference)
import functools

import jax
import jax.numpy as jnp
from jax import lax
from jax.experimental import pallas as pl
from jax.experimental.pallas import tpu as pltpu
from jax.experimental.pallas import tpu_sc as plsc

NUM_CORES = 2
NUM_SUBCORES = 16
NTILES = NUM_CORES * NUM_SUBCORES
LANES = 16
SEQ = 16384
NWORDS_TOTAL = 1000000
EMB = 64
NTAGS = 1000
HIST = 1 << 20
HIST_PER_TILE = HIST // NTILES
ZERO_UNROLL = 16
COUNT_UNROLL = 8
BLK = 65536
NBLK = -(-NWORDS_TOTAL // BLK)


def _sc_histogram(nwords_i32):
    mesh = plsc.VectorSubcoreMesh(core_axis_name="c", subcore_axis_name="s")

    @functools.partial(
        pl.kernel,
        out_type=jax.ShapeDtypeStruct((HIST,), jnp.float32),
        mesh=mesh,
        scratch_types=[
            pltpu.VMEM((SEQ,), jnp.int32),
            pltpu.VMEM((HIST_PER_TILE,), jnp.float32),
            pltpu.SemaphoreType.DMA,
        ],
        compiler_params=pltpu.CompilerParams(needs_layout_passes=False),
    )
    def k(idx_hbm, out_hbm, idx_v, hist_v, sem):
        wid = lax.axis_index("s") * NUM_CORES + lax.axis_index("c")
        cp = pltpu.async_copy(idx_hbm, idx_v, sem)

        zeros16 = jnp.zeros((LANES,), jnp.float32)

        def zero_body(i, carry):
            for u in range(ZERO_UNROLL):
                hist_v[pl.ds((i * ZERO_UNROLL + u) * LANES, LANES)] = zeros16
            return carry

        lax.fori_loop(0, HIST_PER_TILE // (LANES * ZERO_UNROLL), zero_body, 0)
        cp.wait()

        base_w = wid * HIST_PER_TILE
        ones16 = jnp.ones((LANES,), jnp.float32)

        def count_body(g, carry):
            for u in range(COUNT_UNROLL):
                rel = idx_v[pl.ds((g * COUNT_UNROLL + u) * LANES, LANES)] - base_w
                mask = (rel >= 0) & (rel < HIST_PER_TILE)
                plsc.addupdate_scatter(hist_v, [rel], ones16, mask=mask)
            return carry

        lax.fori_loop(0, SEQ // (LANES * COUNT_UNROLL), count_body, 0)
        pltpu.sync_copy(hist_v, out_hbm.at[pl.ds(base_w, HIST_PER_TILE)])

    return k(nwords_i32)


def _tc_matvec_head(table_t, hist, lin_weight, lin_bias2d):
    def body(t_ref, h_ref, w_ref, b_ref, o_ref, acc_ref):
        step = pl.program_id(0)

        @pl.when(step == 0)
        def _():
            acc_ref[...] = jnp.zeros((1, EMB), jnp.float32)

        h = h_ref[...]
        t = t_ref[...]

        @pl.when(step == NBLK - 1)
        def _():
            lane = lax.broadcasted_iota(jnp.int32, (1, BLK), 1)
            limit = NWORDS_TOTAL - (NBLK - 1) * BLK
            acc_ref[...] += lax.dot_general(
                h, jnp.where(lane < limit, t, 0.0),
                dimension_numbers=(((1,), (1,)), ((), ())),
                preferred_element_type=jnp.float32,
            )

        @pl.when(step < NBLK - 1)
        def _():
            acc_ref[...] += lax.dot_general(
                h, t,
                dimension_numbers=(((1,), (1,)), ((), ())),
                preferred_element_type=jnp.float32,
            )

        @pl.when(step == NBLK - 1)
        def _():
            o_ref[...] = (
                lax.dot_general(
                    acc_ref[...], w_ref[...],
                    dimension_numbers=(((1,), (1,)), ((), ())),
                    preferred_element_type=jnp.float32,
                )
                + b_ref[...]
            )

    return pl.pallas_call(
        body,
        grid=(NBLK,),
        in_specs=[
            pl.BlockSpec((EMB, BLK), lambda k: (0, k)),
            pl.BlockSpec((1, BLK), lambda k: (0, k)),
            pl.BlockSpec((NTAGS, EMB), lambda k: (0, 0)),
            pl.BlockSpec((1, NTAGS), lambda k: (0, 0)),
        ],
        out_specs=pl.BlockSpec((1, NTAGS), lambda k: (0, 0)),
        scratch_shapes=[pltpu.VMEM((1, EMB), jnp.float32)],
        out_shape=jax.ShapeDtypeStruct((1, NTAGS), jnp.float32),
    )(table_t, hist, lin_weight, lin_bias2d)


def kernel(nwords, emb_weight, lin_weight, lin_bias):
    idx = nwords.astype(jnp.int32)
    table_t = emb_weight.T
    hist = _sc_histogram(idx).reshape(1, HIST)
    return _tc_matvec_head(table_t, hist, lin_weight,
                           lin_bias.reshape(1, NTAGS))
